# Initial kernel scaffold; baseline (speedup 1.0000x reference)
import jax
import jax.numpy as jnp
from jax.experimental import pallas as pl


def kernel(state_node_names, state_edge_index, state_edge_attr, state_batch, goal_node_names, goal_edge_index, goal_edge_attr, goal_batch, depth, params):
    raise NotImplementedError("write your pallas kernel here")


# speedup vs baseline: 3.1695x; 3.1695x over previous
import functools

import jax
import jax.numpy as jnp
from jax import lax
from jax.experimental import pallas as pl
from jax.experimental.pallas import tpu as pltpu
from jax.experimental.pallas import tpu_sc as plsc

_TWO48 = float(2 ** 48 - 1)
_N = 10000
_E = 320000
_G = 64
_NP = 10240
_NC = 2
_NS = 16
_NW = _NC * _NS
_EW = _E // _NW
_B = 80
_BR = 2048


def _bcast(vec16, i):
    return vec16.at[jnp.full((16,), i, jnp.int32)].get(mode="promise_in_bounds")


def _make_edge_kernel_r1():
    D, CH = 64, 4
    nblk = _EW // _B
    rows_per_sub = _NP // _NS
    mesh = plsc.VectorSubcoreMesh(core_axis_name="c", subcore_axis_name="s", num_cores=_NC, num_subcores=_NS)

    @functools.partial(
        pl.kernel,
        out_type=jax.ShapeDtypeStruct((_NC * _NP, D), jnp.float32),
        mesh=mesh,
        compiler_params=pltpu.CompilerParams(needs_layout_passes=False),
        scratch_types=[
            pltpu.VMEM((_B,), jnp.int32),
            pltpu.VMEM((_B,), jnp.int32),
            pltpu.VMEM((_B,), jnp.float32),
            pltpu.VMEM((_B, D), jnp.float32),
            pltpu.VMEM((D,), jnp.float32),
            pltpu.VMEM((D,), jnp.float32),
            pltpu.VMEM((D,), jnp.float32),
            pltpu.VMEM((_NP,), jnp.float32),
            pltpu.VMEM_SHARED((_NP, D), jnp.float32),
        ],
    )
    def edge_kernel(names, src, dst, a, vv, p, w, out, idx_s, idx_d, a_v,
                    rows, v_v, p_v, w_v, raw_v, acc):
        cid = lax.axis_index("c")
        sid = lax.axis_index("s")
        wid = sid * _NC + cid

        pltpu.sync_copy(vv, v_v)
        pltpu.sync_copy(p, p_v)
        pltpu.sync_copy(w, w_v)
        pltpu.sync_copy(names, raw_v)
        vvec = [v_v[pl.ds(c * 16, 16)] for c in range(CH)]
        pvec = [p_v[pl.ds(c * 16, 16)] for c in range(CH)]
        wvec = [w_v[pl.ds(c * 16, 16)] for c in range(CH)]

        inv = jnp.float32(1.0 / _TWO48)

        def scale_body(k, carry):
            sl = pl.ds(k * 16, 16)
            raw_v[sl] = jnp.clip(raw_v[sl] * inv, 0.0, 1.0)
            return carry

        lax.fori_loop(0, _NP // 16, scale_body, 0)

        zero = jnp.zeros((16,), jnp.float32)
        for i in range(_B):
            for c in range(CH):
                rows[i, pl.ds(c * 16, 16)] = zero
        base_r = sid * rows_per_sub
        for k in range(rows_per_sub // _B):
            pltpu.sync_copy(rows, acc.at[pl.ds(base_r + k * _B, _B)])
        plsc.subcore_barrier()

        ebase = wid * _EW

        def block_body(b, carry):
            base = ebase + b * _B
            pltpu.sync_copy(src.at[pl.ds(base, _B)], idx_s)
            pltpu.sync_copy(dst.at[pl.ds(base, _B)], idx_d)
            pltpu.sync_copy(a.at[pl.ds(base, _B)], a_v)
            for g in range(_B // 16):
                idx16 = idx_s[pl.ds(g * 16, 16)]
                r16 = plsc.load_gather(raw_v, [idx16])
                a16 = a_v[pl.ds(g * 16, 16)]
                for i in range(16):
                    rb = _bcast(r16, i)
                    ab = _bcast(a16, i)
                    e = g * 16 + i
                    for c in range(CH):
                        rows[e, pl.ds(c * 16, 16)] = jnp.maximum(
                            rb * vvec[c] + ab * pvec[c] + wvec[c], 0.0)
            pltpu.sync_copy(rows, acc.at[idx_d], add=True)
            return carry

        lax.fori_loop(0, nblk, block_body, 0)
        plsc.subcore_barrier()
        pltpu.sync_copy(acc.at[pl.ds(base_r, rows_per_sub)],
                        out.at[pl.ds(cid * _NP + base_r, rows_per_sub)])

    return edge_kernel


def _make_edge_kernel_full():
    D, CH = 128, 8
    nblk = _EW // _B
    rows_per_sub = _NP // _NS
    mesh = plsc.VectorSubcoreMesh(core_axis_name="c", subcore_axis_name="s", num_cores=_NC, num_subcores=_NS)

    @functools.partial(
        pl.kernel,
        out_type=jax.ShapeDtypeStruct((_NC * _NP, D), jnp.float32),
        mesh=mesh,
        scratch_types=[
            pltpu.VMEM((_B,), jnp.int32),
            pltpu.VMEM((_B,), jnp.int32),
            pltpu.VMEM((_B,), jnp.float32),
            pltpu.VMEM((_B, D), jnp.float32),
            pltpu.VMEM((D,), jnp.float32),
            pltpu.VMEM((D,), jnp.float32),
            pltpu.VMEM_SHARED((_NP, D), jnp.float32),
        ],
    )
    def edge_kernel(table, src, dst, a, p, q, out, idx_s, idx_d, a_v, rows,
                    p_v, q_v, acc):
        cid = lax.axis_index("c")
        sid = lax.axis_index("s")
        wid = sid * _NC + cid

        pltpu.sync_copy(p, p_v)
        pltpu.sync_copy(q, q_v)
        pv = [p_v[pl.ds(c * 16, 16)] for c in range(CH)]
        qv = [q_v[pl.ds(c * 16, 16)] for c in range(CH)]

        zero = jnp.zeros((16,), jnp.float32)
        for i in range(_B):
            for c in range(CH):
                rows[i, pl.ds(c * 16, 16)] = zero
        base_r = sid * rows_per_sub
        for k in range(rows_per_sub // _B):
            pltpu.sync_copy(rows, acc.at[pl.ds(base_r + k * _B, _B)])
        plsc.subcore_barrier()

        ebase = wid * _EW

        def block_body(b, carry):
            base = ebase + b * _B
            pltpu.sync_copy(src.at[pl.ds(base, _B)], idx_s)
            pltpu.sync_copy(dst.at[pl.ds(base, _B)], idx_d)
            pltpu.sync_copy(a.at[pl.ds(base, _B)], a_v)
            pltpu.sync_copy(table.at[idx_s], rows)
            for g in range(_B // 16):
                a16 = a_v[pl.ds(g * 16, 16)]
                for i in range(16):
                    ab = _bcast(a16, i)
                    e = g * 16 + i
                    for c in range(CH):
                        sl = pl.ds(c * 16, 16)
                        rows[e, sl] = jnp.maximum(
                            rows[e, sl] + ab * pv[c] + qv[c], 0.0)
            pltpu.sync_copy(rows, acc.at[idx_d], add=True)
            return carry

        lax.fori_loop(0, nblk, block_body, 0)
        plsc.subcore_barrier()
        pltpu.sync_copy(acc.at[pl.ds(base_r, rows_per_sub)],
                        out.at[pl.ds(cid * _NP + base_r, rows_per_sub)])

    return edge_kernel


_edge_kernel_r1 = _make_edge_kernel_r1()
_edge_kernel_full = _make_edge_kernel_full()


def _tc1_body(part_ref, nf_ref, v_ref, d_ref, w1_ref, b1_ref, w2_ref, b2_ref,
              out_ref, x0_ref):
    inv = jnp.float32(1.0 / _TWO48)
    x0 = jnp.clip(nf_ref[...] * inv, 0.0, 1.0) * v_ref[...] + d_ref[...]
    h = part_ref[0] + part_ref[1] + x0
    h1 = jnp.maximum(
        jnp.dot(h, w1_ref[...], preferred_element_type=jnp.float32, precision=lax.Precision.HIGHEST)
        + b1_ref[...], 0.0)
    out_ref[...] = jnp.maximum(
        jnp.dot(h1, w2_ref[...], preferred_element_type=jnp.float32, precision=lax.Precision.HIGHEST)
        + b2_ref[...], 0.0)
    x0_ref[...] = x0


def _tc1(part, nf, v, d, w1, b1, w2, b2):
    grid = (_NP // _BR,)
    return pl.pallas_call(
        _tc1_body,
        grid=grid,
        in_specs=[
            pl.BlockSpec((2, _BR, 64), lambda i: (0, i, 0)),
            pl.BlockSpec((_BR, 1), lambda i: (i, 0)),
            pl.BlockSpec((1, 64), lambda i: (0, 0)),
            pl.BlockSpec((1, 64), lambda i: (0, 0)),
            pl.BlockSpec((64, 128), lambda i: (0, 0)),
            pl.BlockSpec((1, 128), lambda i: (0, 0)),
            pl.BlockSpec((128, 128), lambda i: (0, 0)),
            pl.BlockSpec((1, 128), lambda i: (0, 0)),
        ],
        out_specs=(pl.BlockSpec((_BR, 128), lambda i: (i, 0)),
                   pl.BlockSpec((_BR, 64), lambda i: (i, 0))),
        out_shape=(jax.ShapeDtypeStruct((_NP, 128), jnp.float32),
                   jax.ShapeDtypeStruct((_NP, 64), jnp.float32)),
    )(part, nf, v, d, w1, b1, w2, b2)


def _tc2_body(part_ref, x1_ref, w1_ref, b1_ref, w2_ref, b2_ref, pb_ref,
              sum_ref, cnt_ref):
    i = pl.program_id(0)
    h = part_ref[0] + part_ref[1] + x1_ref[...]
    h1 = jnp.maximum(
        jnp.dot(h, w1_ref[...], preferred_element_type=jnp.float32, precision=lax.Precision.HIGHEST)
        + b1_ref[...], 0.0)
    rows = jnp.maximum(
        jnp.dot(h1, w2_ref[...], preferred_element_type=jnp.float32, precision=lax.Precision.HIGHEST)
        + b2_ref[...], 0.0)
    gids = lax.broadcasted_iota(jnp.int32, (1, _G), 1)
    onehot = (pb_ref[...] == gids).astype(jnp.float32)
    psum = lax.dot_general(onehot, rows, (((0,), (0,)), ((), ())),
                           preferred_element_type=jnp.float32, precision=lax.Precision.HIGHEST)
    pcnt = lax.dot_general(onehot, jnp.ones_like(rows),
                           (((0,), (0,)), ((), ())),
                           preferred_element_type=jnp.float32, precision=lax.Precision.HIGHEST)

    @pl.when(i == 0)
    def _():
        sum_ref[...] = jnp.zeros_like(sum_ref)
        cnt_ref[...] = jnp.zeros_like(cnt_ref)

    sum_ref[...] += psum
    cnt_ref[...] += pcnt


def _tc2(part, x1, w1, b1, w2, b2, pb):
    grid = (_NP // _BR,)
    return pl.pallas_call(
        _tc2_body,
        grid=grid,
        in_specs=[
            pl.BlockSpec((2, _BR, 128), lambda i: (0, i, 0)),
            pl.BlockSpec((_BR, 128), lambda i: (i, 0)),
            pl.BlockSpec((128, 128), lambda i: (0, 0)),
            pl.BlockSpec((1, 128), lambda i: (0, 0)),
            pl.BlockSpec((128, 128), lambda i: (0, 0)),
            pl.BlockSpec((1, 128), lambda i: (0, 0)),
            pl.BlockSpec((_BR, 1), lambda i: (i, 0)),
        ],
        out_specs=(pl.BlockSpec((_G, 128), lambda i: (0, 0)),
                   pl.BlockSpec((_G, 128), lambda i: (0, 0))),
        out_shape=(jax.ShapeDtypeStruct((_G, 128), jnp.float32),
                   jax.ShapeDtypeStruct((_G, 128), jnp.float32)),
    )(part, x1, w1, b1, w2, b2, pb)


def _tc3_body(ps_ref, cs_ref, pg_ref, cg_ref, dep_ref, ws_ref, wg_ref,
              wd_ref, rb_ref,
              b0_ref, b1_ref, b2_ref, b3_ref, b4_ref, b5_ref, b6_ref, b7_ref,
              c0_ref, c1_ref, c2_ref, c3_ref, c4_ref, c5_ref, c6_ref, c7_ref,
              d0_ref, d1_ref, d2_ref, d3_ref, d4_ref, d5_ref, d6_ref, d7_ref,
              wo_ref, bo_ref, out_ref):
    s = ps_ref[...] / jnp.maximum(cs_ref[...], 1.0)
    g = pg_ref[...] / jnp.maximum(cg_ref[...], 1.0)
    h = jnp.maximum(
        jnp.dot(s, ws_ref[...], preferred_element_type=jnp.float32, precision=lax.Precision.HIGHEST)
        + jnp.dot(g, wg_ref[...], preferred_element_type=jnp.float32, precision=lax.Precision.HIGHEST)
        + dep_ref[...] * wd_ref[...] + rb_ref[...], 0.0)

    blocks = [
        (b0_ref, b1_ref, b2_ref, b3_ref, b4_ref, b5_ref, b6_ref, b7_ref),
        (c0_ref, c1_ref, c2_ref, c3_ref, c4_ref, c5_ref, c6_ref, c7_ref),
        (d0_ref, d1_ref, d2_ref, d3_ref, d4_ref, d5_ref, d6_ref, d7_ref),
    ]
    for (l1w, l1b, g1, bt1, l2w, l2b, g2, bt2) in blocks:
        o = jnp.dot(h, l1w[...], preferred_element_type=jnp.float32, precision=lax.Precision.HIGHEST) + l1b[...]
        m = jnp.mean(o, axis=0, keepdims=True)
        v = jnp.mean((o - m) ** 2, axis=0, keepdims=True)
        o = jnp.maximum((o - m) / jnp.sqrt(v + 1e-5) * g1[...] + bt1[...], 0.0)
        o = jnp.dot(o, l2w[...], preferred_element_type=jnp.float32, precision=lax.Precision.HIGHEST) + l2b[...]
        m = jnp.mean(o, axis=0, keepdims=True)
        v = jnp.mean((o - m) ** 2, axis=0, keepdims=True)
        o = (o - m) / jnp.sqrt(v + 1e-5) * g2[...] + bt2[...]
        h = jnp.maximum(o + h, 0.0)

    logit = jnp.dot(h, wo_ref[...], preferred_element_type=jnp.float32, precision=lax.Precision.HIGHEST) \
        + bo_ref[...]
    out_ref[...] = jnp.clip(1.0 / (1.0 + jnp.exp(-logit)), 0.001, 0.999)


def _tc3(args):
    return pl.pallas_call(
        _tc3_body,
        out_shape=jax.ShapeDtypeStruct((_G, 1), jnp.float32),
    )(*args)


def kernel(state_node_names, state_edge_index, state_edge_attr, state_batch,
           goal_node_names, goal_edge_index, goal_edge_attr, goal_batch,
           depth, params):
    pr = params
    v = jnp.maximum(pr['id_w1'][0], 0.0) @ pr['id_w2']
    d = pr['id_b2']
    u = jnp.maximum(pr['e_w1'][0], 0.0) @ pr['e_w2']
    cvec = pr['e_b2']

    def side(names, ei, ea, batch, c1, c2):
        nf = jnp.pad(names.astype(jnp.float32), (0, _NP - _N))
        src, dst = ei[0], ei[1]
        a = ea[:, 0]
        p1 = u @ c1['lin_w']
        w1 = cvec @ c1['lin_w'] + c1['lin_b'] + d
        p2 = u @ c2['lin_w']
        q2 = cvec @ c2['lin_w'] + c2['lin_b']
        pb = jnp.pad(batch, (0, _NP - _N), constant_values=_G).reshape(_NP, 1)
        agg1 = _edge_kernel_r1(nf, src, dst, a, v, p1, w1)
        x1, x0 = _tc1(agg1.reshape(_NC, _NP, 64), nf.reshape(_NP, 1),
                      v.reshape(1, 64), d.reshape(1, 64),
                      c1['n_w1'], c1['n_b1'].reshape(1, 128),
                      c1['n_w2'], c1['n_b2'].reshape(1, 128))
        agg2 = _edge_kernel_full(x1, src, dst, a, p2, q2)
        return _tc2(agg2.reshape(_NC, _NP, 128), x1,
                    c2['n_w1'], c2['n_b1'].reshape(1, 128),
                    c2['n_w2'], c2['n_b2'].reshape(1, 128), pb)

    sum_s, cnt_s = side(state_node_names, state_edge_index, state_edge_attr,
                        state_batch, pr['state_conv1'], pr['state_conv2'])
    sum_g, cnt_g = side(goal_node_names, goal_edge_index, goal_edge_attr,
                        goal_batch, pr['goal_conv1'], pr['goal_conv2'])

    riw = pr['reg_in_w']
    args = [sum_s, cnt_s, sum_g, cnt_g, depth.reshape(_G, 1),
            riw[:128], riw[128:256], riw[256].reshape(1, 128),
            pr['reg_in_b'].reshape(1, 128)]
    for blk in pr['blocks']:
        args += [blk['l1_w'], blk['l1_b'].reshape(1, 128),
                 blk['g1'].reshape(1, 128), blk['bt1'].reshape(1, 128),
                 blk['l2_w'], blk['l2_b'].reshape(1, 128),
                 blk['g2'].reshape(1, 128), blk['bt2'].reshape(1, 128)]
    args += [pr['reg_out_w'], pr['reg_out_b'].reshape(1, 1)]

    out = _tc3(args)
    return out.reshape(_G)
